# baseline (speedup 1.0000x reference)
import numpy as np
import jax
import jax.numpy as jnp
from jax.experimental import pallas as pl
from jax.experimental.pallas import tpu as pltpu

_N = 100000
_V = 512
_D = 128
_SQD = np.float32(np.sqrt(_D))
_SQAVG = np.float32(np.sqrt(_N / _V))
_SQ2 = np.float32(np.sqrt(2.0))


def _silu_cst():
    z = np.random.RandomState(0).randn(1_000_000).astype(np.float64)
    s = z / (1.0 + np.exp(-z))
    return np.float32(1.0 / np.sqrt(np.mean(s * s)))


_CST = _silu_cst()

_B = 10000
_NB = _N // _B


def _seg_body(x_ref, pos_ref, b_ref, seg_ref):
    i = pl.program_id(0)
    b = b_ref[0].astype(jnp.int16)
    p = pos_ref[0].astype(jnp.bfloat16)
    iota = jax.lax.broadcasted_iota(jnp.int16, (_V, _B), 0)
    wT = jnp.where(iota == b, p, jnp.bfloat16(0.0))
    part = jax.lax.dot_general(wT, x_ref[...].astype(jnp.bfloat16),
                               (((1,), (0,)), ((), ())),
                               preferred_element_type=jnp.float32)

    @pl.when(i == 0)
    def _():
        seg_ref[...] = part

    @pl.when(i > 0)
    def _():
        seg_ref[...] += part


def _virt_body(xv_ref, seg_ref, wvsc_ref, wtp_ref, wn2v_ref,
               xvout_ref, tab_ref):
    s_virtual = jnp.dot(xv_ref[...], wvsc_ref[...],
                        preferred_element_type=jnp.float32) / _SQD
    m = jnp.dot(seg_ref[...], wtp_ref[...],
                preferred_element_type=jnp.float32) / (_SQD * _SQAVG)
    m = _CST * m * jax.nn.sigmoid(m)
    xv_out = (s_virtual + m) / _SQ2
    xvout_ref[...] = xv_out
    t = jnp.dot(xv_out, wn2v_ref[...],
                preferred_element_type=jnp.float32) / _SQD
    tab_ref[...] = (_CST / _SQ2) * t * jax.nn.sigmoid(t)


def _node_body(x_ref, b_ref, tab_ref, wnsc_ref, out_ref):
    b = b_ref[0].astype(jnp.int16)
    iota = jax.lax.broadcasted_iota(jnp.int16, (_V, _B), 0)
    ohT = jnp.where(iota == b, jnp.bfloat16(1.0), jnp.bfloat16(0.0))
    gathered = jax.lax.dot_general(ohT, tab_ref[...].astype(jnp.bfloat16),
                                   (((0,), (0,)), ((), ())),
                                   preferred_element_type=jnp.float32)
    s_node = jax.lax.dot_general(x_ref[...].astype(jnp.bfloat16),
                                 wnsc_ref[...].astype(jnp.bfloat16),
                                 (((1,), (0,)), ((), ())),
                                 preferred_element_type=jnp.float32)
    out_ref[...] = s_node + gathered


def kernel(x_virtual, x_node, node_pos_sh, batch, W_vsc, W_nsc, W_tp, W_n2v):
    b3 = batch.astype(jnp.int32).reshape(_NB, 1, _B)
    p3 = node_pos_sh.astype(jnp.float32).reshape(_NB, 1, _B)
    wnsc_s = W_nsc * np.float32(1.0 / (_SQD * _SQ2))

    seg = pl.pallas_call(
        _seg_body,
        grid=(_NB,),
        in_specs=[
            pl.BlockSpec((_B, _D), lambda i: (i, 0)),
            pl.BlockSpec((1, 1, _B), lambda i: (i, 0, 0)),
            pl.BlockSpec((1, 1, _B), lambda i: (i, 0, 0)),
        ],
        out_specs=pl.BlockSpec((_V, _D), lambda i: (0, 0)),
        out_shape=jax.ShapeDtypeStruct((_V, _D), jnp.float32),
    )(x_node, p3, b3)

    xv_out, tab = pl.pallas_call(
        _virt_body,
        out_shape=(jax.ShapeDtypeStruct((_V, _D), jnp.float32),
                   jax.ShapeDtypeStruct((_V, _D), jnp.float32)),
    )(x_virtual, seg, W_vsc, W_tp, W_n2v)

    x_node_out = pl.pallas_call(
        _node_body,
        grid=(_NB,),
        in_specs=[
            pl.BlockSpec((_B, _D), lambda i: (i, 0)),
            pl.BlockSpec((1, 1, _B), lambda i: (i, 0, 0)),
            pl.BlockSpec((_V, _D), lambda i: (0, 0)),
            pl.BlockSpec((_D, _D), lambda i: (0, 0)),
        ],
        out_specs=pl.BlockSpec((_B, _D), lambda i: (i, 0)),
        out_shape=jax.ShapeDtypeStruct((_N, _D), jnp.float32),
    )(x_node, b3, tab, wnsc_s)

    return (xv_out, x_node_out)


# speedup vs baseline: 1.0426x; 1.0426x over previous
import numpy as np
import jax
import jax.numpy as jnp
from jax.experimental import pallas as pl
from jax.experimental.pallas import tpu as pltpu

_N = 100000
_V = 512
_D = 128
_SQD = np.float32(np.sqrt(_D))
_SQAVG = np.float32(np.sqrt(_N / _V))
_SQ2 = np.float32(np.sqrt(2.0))


def _silu_cst():
    z = np.random.RandomState(0).randn(1_000_000).astype(np.float64)
    s = z / (1.0 + np.exp(-z))
    return np.float32(1.0 / np.sqrt(np.mean(s * s)))


_CST = _silu_cst()

_B = 10000
_NB = _N // _B


def _seg_body(x_ref, pos_ref, b_ref, seg_ref):
    i = pl.program_id(0)
    b = b_ref[0].astype(jnp.int16)
    p = pos_ref[0].astype(jnp.bfloat16)
    iota = jax.lax.broadcasted_iota(jnp.int16, (_V, _B), 0)
    wT = jnp.where(iota == b, p, jnp.bfloat16(0.0))
    part = jax.lax.dot_general(wT, x_ref[...].astype(jnp.bfloat16),
                               (((1,), (0,)), ((), ())),
                               preferred_element_type=jnp.float32)

    @pl.when(i == 0)
    def _():
        seg_ref[...] = part

    @pl.when(i > 0)
    def _():
        seg_ref[...] += part


def _virt_body(xv_ref, seg_ref, wvsc_ref, wtp_ref, wn2v_ref,
               xvout_ref, tab_ref):
    s_virtual = jnp.dot(xv_ref[...], wvsc_ref[...],
                        preferred_element_type=jnp.float32) / _SQD
    m = jnp.dot(seg_ref[...], wtp_ref[...],
                preferred_element_type=jnp.float32) / (_SQD * _SQAVG)
    m = _CST * m * jax.nn.sigmoid(m)
    xv_out = (s_virtual + m) / _SQ2
    xvout_ref[...] = xv_out
    t = jnp.dot(xv_out, wn2v_ref[...],
                preferred_element_type=jnp.float32) / _SQD
    tab_ref[...] = (_CST / _SQ2) * t * jax.nn.sigmoid(t)


def _node_body(x_ref, b_ref, tab_ref, wnsc_ref, out_ref):
    b = b_ref[0].astype(jnp.int16)
    iota = jax.lax.broadcasted_iota(jnp.int16, (_V, _B), 0)
    ohT = jnp.where(iota == b, jnp.bfloat16(1.0), jnp.bfloat16(0.0))
    gathered = jax.lax.dot_general(ohT, tab_ref[...].astype(jnp.bfloat16),
                                   (((0,), (0,)), ((), ())),
                                   preferred_element_type=jnp.float32)
    s_node = jnp.dot(x_ref[...], wnsc_ref[...],
                     preferred_element_type=jnp.float32)
    out_ref[...] = s_node + gathered


def kernel(x_virtual, x_node, node_pos_sh, batch, W_vsc, W_nsc, W_tp, W_n2v):
    b3 = batch.astype(jnp.int32).reshape(_NB, 1, _B)
    p3 = node_pos_sh.astype(jnp.float32).reshape(_NB, 1, _B)
    wnsc_s = W_nsc * np.float32(1.0 / (_SQD * _SQ2))

    seg = pl.pallas_call(
        _seg_body,
        grid=(_NB,),
        in_specs=[
            pl.BlockSpec((_B, _D), lambda i: (i, 0)),
            pl.BlockSpec((1, 1, _B), lambda i: (i, 0, 0)),
            pl.BlockSpec((1, 1, _B), lambda i: (i, 0, 0)),
        ],
        out_specs=pl.BlockSpec((_V, _D), lambda i: (0, 0)),
        out_shape=jax.ShapeDtypeStruct((_V, _D), jnp.float32),
    )(x_node, p3, b3)

    xv_out, tab = pl.pallas_call(
        _virt_body,
        out_shape=(jax.ShapeDtypeStruct((_V, _D), jnp.float32),
                   jax.ShapeDtypeStruct((_V, _D), jnp.float32)),
    )(x_virtual, seg, W_vsc, W_tp, W_n2v)

    x_node_out = pl.pallas_call(
        _node_body,
        grid=(_NB,),
        in_specs=[
            pl.BlockSpec((_B, _D), lambda i: (i, 0)),
            pl.BlockSpec((1, 1, _B), lambda i: (i, 0, 0)),
            pl.BlockSpec((_V, _D), lambda i: (0, 0)),
            pl.BlockSpec((_D, _D), lambda i: (0, 0)),
        ],
        out_specs=pl.BlockSpec((_B, _D), lambda i: (i, 0)),
        out_shape=jax.ShapeDtypeStruct((_N, _D), jnp.float32),
    )(x_node, b3, tab, wnsc_s)

    return (xv_out, x_node_out)


# speedup vs baseline: 1.0610x; 1.0176x over previous
import numpy as np
import jax
import jax.numpy as jnp
from jax.experimental import pallas as pl

_N = 100000
_V = 512
_D = 128
_SQD = np.float32(np.sqrt(_D))
_SQAVG = np.float32(np.sqrt(_N / _V))
_SQ2 = np.float32(np.sqrt(2.0))


def _silu_cst():
    z = np.random.RandomState(0).randn(1_000_000).astype(np.float64)
    s = z / (1.0 + np.exp(-z))
    return np.float32(1.0 / np.sqrt(np.mean(s * s)))


_CST = _silu_cst()

_B = 10000
_NB = _N // _B


def _seg_body(x_ref, pos_ref, b_ref, seg_ref):
    i = pl.program_id(0)
    b = b_ref[0].astype(jnp.int16)
    p = pos_ref[0].astype(jnp.bfloat16)
    iota = jax.lax.broadcasted_iota(jnp.int16, (_V, _B), 0)
    wT = jnp.where(iota == b, p, jnp.bfloat16(0.0))
    part = jax.lax.dot_general(wT, x_ref[...].astype(jnp.bfloat16),
                               (((1,), (0,)), ((), ())),
                               preferred_element_type=jnp.float32)

    @pl.when(i == 0)
    def _():
        seg_ref[...] = part

    @pl.when(i > 0)
    def _():
        seg_ref[...] += part


def _virt_body(xv_ref, seg_ref, wvsc_ref, wtp_ref, wn2v_ref,
               xvout_ref, tab_ref):
    s_virtual = jnp.dot(xv_ref[...], wvsc_ref[...],
                        preferred_element_type=jnp.float32) / _SQD
    m = jnp.dot(seg_ref[...], wtp_ref[...],
                preferred_element_type=jnp.float32) / (_SQD * _SQAVG)
    m = _CST * m * jax.nn.sigmoid(m)
    xv_out = (s_virtual + m) / _SQ2
    xvout_ref[...] = xv_out
    t = jnp.dot(xv_out, wn2v_ref[...],
                preferred_element_type=jnp.float32) / _SQD
    tab_ref[...] = _CST * t * jax.nn.sigmoid(t)


def _node_body(x_ref, b_ref, tab_ref, wnsc_ref, out_ref):
    b = b_ref[0].astype(jnp.int16)
    iota = jax.lax.broadcasted_iota(jnp.int16, (_V, _B), 0)
    ohT = jnp.where(iota == b, jnp.bfloat16(1.0), jnp.bfloat16(0.0))
    gathered = jax.lax.dot_general(ohT, tab_ref[...].astype(jnp.bfloat16),
                                   (((0,), (0,)), ((), ())),
                                   preferred_element_type=jnp.float32)
    s_node = jnp.dot(x_ref[...], wnsc_ref[...],
                     preferred_element_type=jnp.float32) / _SQD
    out_ref[...] = (s_node + gathered) / _SQ2


def kernel(x_virtual, x_node, node_pos_sh, batch, W_vsc, W_nsc, W_tp, W_n2v):
    b3 = batch.astype(jnp.int32).reshape(_NB, 1, _B)
    p3 = node_pos_sh.astype(jnp.float32).reshape(_NB, 1, _B)

    seg = pl.pallas_call(
        _seg_body,
        grid=(_NB,),
        in_specs=[
            pl.BlockSpec((_B, _D), lambda i: (i, 0)),
            pl.BlockSpec((1, 1, _B), lambda i: (i, 0, 0)),
            pl.BlockSpec((1, 1, _B), lambda i: (i, 0, 0)),
        ],
        out_specs=pl.BlockSpec((_V, _D), lambda i: (0, 0)),
        out_shape=jax.ShapeDtypeStruct((_V, _D), jnp.float32),
    )(x_node, p3, b3)

    xv_out, tab = pl.pallas_call(
        _virt_body,
        out_shape=(jax.ShapeDtypeStruct((_V, _D), jnp.float32),
                   jax.ShapeDtypeStruct((_V, _D), jnp.float32)),
    )(x_virtual, seg, W_vsc, W_tp, W_n2v)

    x_node_out = pl.pallas_call(
        _node_body,
        grid=(_NB,),
        in_specs=[
            pl.BlockSpec((_B, _D), lambda i: (i, 0)),
            pl.BlockSpec((1, 1, _B), lambda i: (i, 0, 0)),
            pl.BlockSpec((_V, _D), lambda i: (0, 0)),
            pl.BlockSpec((_D, _D), lambda i: (0, 0)),
        ],
        out_specs=pl.BlockSpec((_B, _D), lambda i: (i, 0)),
        out_shape=jax.ShapeDtypeStruct((_N, _D), jnp.float32),
    )(x_node, b3, tab, W_nsc)

    return (xv_out, x_node_out)
